# baseline (speedup 1.0000x reference)
import functools

import jax
import jax.numpy as jnp
from jax.experimental import pallas as pl
from jax.experimental.pallas import tpu as pltpu

_BATCH = 16
_NCLS = 80
_H = 128
_W = 128
_TOPK = 100
_SCALE = 4.0
_THRESH = 0.01


def _nms_body(hm_ref, out_ref):
    x = hm_ref[...]
    neg = jnp.float32(-jnp.inf)
    up = jnp.concatenate([jnp.full_like(x[:, :1, :], neg), x[:, :-1, :]], axis=1)
    dn = jnp.concatenate([x[:, 1:, :], jnp.full_like(x[:, :1, :], neg)], axis=1)
    v = jnp.maximum(jnp.maximum(up, dn), x)
    lf = jnp.concatenate([jnp.full_like(v[:, :, :1], neg), v[:, :, :-1]], axis=2)
    rt = jnp.concatenate([v[:, :, 1:], jnp.full_like(v[:, :, :1], neg)], axis=2)
    m = jnp.maximum(jnp.maximum(lf, rt), v)
    out_ref[...] = jnp.where(m == x, x, 0.0)


def _nms_filter(heatmap):
    bc = _BATCH * _NCLS
    hm = heatmap.reshape(bc, _H, _W)
    cb = 80
    grid = (bc // cb,)
    return pl.pallas_call(
        _nms_body,
        grid=grid,
        in_specs=[pl.BlockSpec((cb, _H, _W), lambda i: (i, 0, 0))],
        out_specs=pl.BlockSpec((cb, _H, _W), lambda i: (i, 0, 0)),
        out_shape=jax.ShapeDtypeStruct((bc, _H, _W), jnp.float32),
    )(hm).reshape(_BATCH, _NCLS, _H, _W)


def kernel(heatmap, offset, wh):
    filt = _nms_filter(heatmap)
    flat = filt.reshape(_BATCH, -1)
    scores, indices = jax.lax.top_k(flat, _TOPK)
    hw = _H * _W
    ids = (indices // hw).astype(jnp.float32)[:, :, None]
    spatial = indices % hw
    ys = spatial // _W
    xs = spatial % _W
    b_idx = jnp.arange(_BATCH)[:, None]
    xo = offset[b_idx, 0, ys, xs]
    yo = offset[b_idx, 1, ys, xs]
    w = wh[b_idx, 0, ys, xs]
    h = wh[b_idx, 1, ys, xs]
    cx = xs.astype(jnp.float32) + xo
    cy = ys.astype(jnp.float32) + yo
    bboxes = jnp.stack([cx - w / 2, cy - h / 2, cx + w / 2, cy + h / 2], axis=-1)
    scores = scores[:, :, None]
    mask = scores > _THRESH
    ids = jnp.where(mask, ids, -1.0)
    out_scores = jnp.where(mask, scores, -1.0)
    bboxes = jnp.where(mask, bboxes, -1.0)
    return (ids, out_scores, bboxes * _SCALE)


# speedup vs baseline: 3.5756x; 3.5756x over previous
import functools

import jax
import jax.numpy as jnp
from jax import lax
from jax.experimental import pallas as pl
from jax.experimental.pallas import tpu as pltpu
from jax.experimental.pallas import tpu_sc as plsc

_BATCH = 16
_NCLS = 80
_H = 128
_W = 128
_TOPK = 100
_KPAD = 128
_SCALE = 4.0
_THRESH = 0.01
_HW = _H * _W
_NCAND = _NCLS * (_H // 2) * (_W // 2)


def _nms_reduce_body(hm_ref, val_ref, idx_ref):
    x = hm_ref[...]
    neg = jnp.float32(-jnp.inf)
    up = jnp.concatenate([jnp.full_like(x[:, :1, :], neg), x[:, :-1, :]], axis=1)
    dn = jnp.concatenate([x[:, 1:, :], jnp.full_like(x[:, :1, :], neg)], axis=1)
    v = jnp.maximum(jnp.maximum(up, dn), x)
    lf = jnp.concatenate([jnp.full_like(v[:, :, :1], neg), v[:, :, :-1]], axis=2)
    rt = jnp.concatenate([v[:, :, 1:], jnp.full_like(v[:, :, :1], neg)], axis=2)
    m = jnp.maximum(jnp.maximum(lf, rt), v)
    f = jnp.where(m == x, x, 0.0)
    fr = f.reshape(_NCLS, _H // 2, 2, _W)
    fe = fr[:, :, 0, :].reshape(_NCLS, _H // 2, _W)
    fo = fr[:, :, 1, :].reshape(_NCLS, _H // 2, _W)
    vh = jnp.maximum(fe, fo)
    yy = lax.broadcasted_iota(jnp.int32, (_NCLS, _H // 2, _W), 1) * 2
    yb = jnp.where(fe >= fo, yy, yy + 1)
    neg_col = jnp.full_like(vh[:, :, :1], neg)
    vhs = jnp.concatenate([vh[:, :, 1:], neg_col], axis=2)
    ybs = jnp.concatenate([yb[:, :, 1:], jnp.zeros_like(yb[:, :, :1])], axis=2)
    win = vh >= vhs
    pairmax = jnp.maximum(vh, vhs)
    xl = lax.broadcasted_iota(jnp.int32, (_NCLS, _H // 2, _W), 2)
    xwin = jnp.where(win, xl, xl + 1)
    ywin = jnp.where(win, yb, ybs)
    cls = lax.broadcasted_iota(jnp.int32, (_NCLS, _H // 2, _W), 0)
    idxfull = (cls * _HW + ywin * _W + xwin).astype(jnp.float32)
    sel = (lax.broadcasted_iota(jnp.int32, (_W, _W // 2), 0)
           == 2 * lax.broadcasted_iota(jnp.int32, (_W, _W // 2), 1)
           ).astype(jnp.float32)
    dn_spec = (((2,), (0,)), ((), ()))
    vals = lax.dot_general(pairmax, sel, dn_spec,
                           precision=lax.Precision.HIGHEST)
    idxc = lax.dot_general(idxfull, sel, dn_spec,
                           precision=lax.Precision.HIGHEST)
    val_ref[...] = vals
    idx_ref[...] = idxc.astype(jnp.int32)


def _nms_reduce(heatmap):
    hm = heatmap.reshape(_BATCH * _NCLS, _H, _W)
    grid = (_BATCH,)
    vals, idx = pl.pallas_call(
        _nms_reduce_body,
        grid=grid,
        in_specs=[pl.BlockSpec((_NCLS, _H, _W), lambda i: (i, 0, 0))],
        out_specs=[
            pl.BlockSpec((_NCLS, _H // 2, _W // 2), lambda i: (i, 0, 0)),
            pl.BlockSpec((_NCLS, _H // 2, _W // 2), lambda i: (i, 0, 0)),
        ],
        out_shape=[
            jax.ShapeDtypeStruct((_BATCH * _NCLS, _H // 2, _W // 2), jnp.float32),
            jax.ShapeDtypeStruct((_BATCH * _NCLS, _H // 2, _W // 2), jnp.int32),
        ],
    )(hm)
    return vals.reshape(_BATCH, _NCAND), idx.reshape(_BATCH * _NCAND)


def _decode_body(idxtab_hbm, off_hbm, wh_hbm, pos_hbm, sc_hbm,
                 ids_hbm, sco_hbm, x0_hbm, y0_hbm, x1_hbm, y1_hbm,
                 pos_v, gpos_v, idx_v, sc_v, gidx_v, voff_v, vwh_v,
                 ids_v, sco_v, x0_v, y0_v, x1_v, y1_v, sem):
    wid = lax.axis_index("s") * 2 + lax.axis_index("c")
    b = wid >> 1
    h = wid & 1
    col = h * 64
    base = b * (2 * _HW)
    pltpu.sync_copy(pos_hbm.at[b, pl.ds(col, 64)], pos_v)
    pltpu.sync_copy(sc_hbm.at[b, pl.ds(col, 64)], sc_v)
    pbase = b * _NCAND
    for j in range(4):
        sl = pl.ds(j * 16, 16)
        gpos_v[sl] = pos_v[sl] + pbase
    pltpu.async_copy(idxtab_hbm.at[gpos_v], idx_v, sem).wait()
    for j in range(4):
        sl = pl.ds(j * 16, 16)
        sp = jnp.bitwise_and(idx_v[sl], _HW - 1) + base
        gidx_v[sl] = sp
        gidx_v[pl.ds(64 + j * 16, 16)] = sp + _HW
    pltpu.async_copy(off_hbm.at[gidx_v], voff_v, sem).wait()
    pltpu.async_copy(wh_hbm.at[gidx_v], vwh_v, sem).wait()
    for j in range(4):
        sl = pl.ds(j * 16, 16)
        sh = pl.ds(64 + j * 16, 16)
        idx = idx_v[sl]
        score = sc_v[sl]
        cls = lax.shift_right_logical(idx, 14)
        sp = jnp.bitwise_and(idx, _HW - 1)
        ys = lax.shift_right_logical(sp, 7)
        xs = jnp.bitwise_and(sp, _W - 1)
        off_x = voff_v[sl]
        off_y = voff_v[sh]
        bw = vwh_v[sl]
        bh = vwh_v[sh]
        cx = xs.astype(jnp.float32) + off_x
        cy = ys.astype(jnp.float32) + off_y
        hw2 = bw * 0.5
        hh2 = bh * 0.5
        keep = score > _THRESH
        neg1 = jnp.full((16,), -1.0, jnp.float32)
        neg4 = jnp.full((16,), -_SCALE, jnp.float32)
        ids_v[sl] = jnp.where(keep, cls.astype(jnp.float32), neg1)
        sco_v[sl] = jnp.where(keep, score, neg1)
        x0_v[sl] = jnp.where(keep, (cx - hw2) * _SCALE, neg4)
        y0_v[sl] = jnp.where(keep, (cy - hh2) * _SCALE, neg4)
        x1_v[sl] = jnp.where(keep, (cx + hw2) * _SCALE, neg4)
        y1_v[sl] = jnp.where(keep, (cy + hh2) * _SCALE, neg4)
    pltpu.sync_copy(ids_v, ids_hbm.at[b, pl.ds(col, 64)])
    pltpu.sync_copy(sco_v, sco_hbm.at[b, pl.ds(col, 64)])
    pltpu.sync_copy(x0_v, x0_hbm.at[b, pl.ds(col, 64)])
    pltpu.sync_copy(y0_v, y0_hbm.at[b, pl.ds(col, 64)])
    pltpu.sync_copy(x1_v, x1_hbm.at[b, pl.ds(col, 64)])
    pltpu.sync_copy(y1_v, y1_hbm.at[b, pl.ds(col, 64)])


def _sc_decode(idx_table, offset_flat, wh_flat, pos_pad, sc_pad):
    f32 = jnp.float32
    out = jax.ShapeDtypeStruct((_BATCH, _KPAD), f32)
    fn = functools.partial(
        pl.kernel,
        mesh=plsc.VectorSubcoreMesh(core_axis_name="c", subcore_axis_name="s"),
        out_type=(out,) * 6,
        scratch_types=[
            pltpu.VMEM((64,), jnp.int32),
            pltpu.VMEM((64,), jnp.int32),
            pltpu.VMEM((64,), jnp.int32),
            pltpu.VMEM((64,), f32),
            pltpu.VMEM((128,), jnp.int32),
            pltpu.VMEM((128,), f32),
            pltpu.VMEM((128,), f32),
            pltpu.VMEM((64,), f32),
            pltpu.VMEM((64,), f32),
            pltpu.VMEM((64,), f32),
            pltpu.VMEM((64,), f32),
            pltpu.VMEM((64,), f32),
            pltpu.VMEM((64,), f32),
            pltpu.SemaphoreType.DMA,
        ],
    )(_decode_body)
    return fn(idx_table, offset_flat, wh_flat, pos_pad, sc_pad)


def kernel(heatmap, offset, wh):
    vals, idx_table = _nms_reduce(heatmap)
    scores, pos = jax.lax.top_k(vals, _TOPK)
    pad = _KPAD - _TOPK
    pos_pad = jnp.pad(pos.astype(jnp.int32), ((0, 0), (0, pad)))
    sc_pad = jnp.pad(scores, ((0, 0), (0, pad)), constant_values=-1.0)
    offset_flat = offset.reshape(_BATCH * 2 * _HW)
    wh_flat = wh.reshape(_BATCH * 2 * _HW)
    ids, sco, x0, y0, x1, y1 = _sc_decode(idx_table, offset_flat, wh_flat,
                                          pos_pad, sc_pad)
    ids = ids[:, :_TOPK, None]
    out_scores = sco[:, :_TOPK, None]
    bboxes = jnp.stack([x0[:, :_TOPK], y0[:, :_TOPK], x1[:, :_TOPK], y1[:, :_TOPK]], axis=-1)
    return (ids, out_scores, bboxes)
